# baseline (speedup 1.0000x reference)
import functools

import jax
import jax.numpy as jnp
from jax import lax
from jax.experimental import pallas as pl
from jax.experimental.pallas import tpu as pltpu
from jax.experimental.pallas import tpu_sc as plsc

LANES = 16
NUM_CORES = 2
NUM_SUBCORES = 16
NUM_WORKERS = NUM_CORES * NUM_SUBCORES
IDX_CHUNK = 128

_TAKE_DNUMS = lax.GatherDimensionNumbers(
    offset_dims=(), collapsed_slice_dims=(0,), start_index_map=(0,))


def _take16(x, idx):
    return lax.gather(x, idx[:, None], _TAKE_DNUMS, slice_sizes=(1,),
                      mode=lax.GatherScatterMode.PROMISE_IN_BOUNDS)


@functools.lru_cache(maxsize=None)
def _build(batch: int, dim: int):
    b_per_w = batch // NUM_WORKERS
    n_chunks = b_per_w // IDX_CHUNK
    groups = b_per_w // LANES

    mesh = plsc.VectorSubcoreMesh(core_axis_name="c", subcore_axis_name="s")

    @functools.partial(
        pl.kernel,
        mesh=mesh,
        out_type=jax.ShapeDtypeStruct((batch,), jnp.float32),
        compiler_params=pltpu.CompilerParams(use_tc_tiling_on_sc=False),
        scratch_types=[
            pltpu.VMEM((n_chunks, IDX_CHUNK), jnp.int32),
            pltpu.VMEM((n_chunks, IDX_CHUNK), jnp.int32),
            pltpu.VMEM((b_per_w, dim), jnp.float32),
            pltpu.VMEM((b_per_w, dim), jnp.float32),
            pltpu.VMEM((b_per_w * dim,), jnp.float32),
            pltpu.VMEM((b_per_w,), jnp.float32),
            pltpu.SemaphoreType.DMA,
        ],
    )
    def sc_kernel(uids_hbm, cids_hbm, umat_hbm, imat_hbm, out_hbm,
                  uid_v, cid_v, urows_v, crows_v, prod_v, out_v, sem):
        wid = lax.axis_index("s") * NUM_CORES + lax.axis_index("c")
        base = wid * b_per_w

        for j in range(n_chunks):
            pltpu.sync_copy(uids_hbm.at[pl.ds(base + j * IDX_CHUNK, IDX_CHUNK)],
                            uid_v.at[j])
            pltpu.sync_copy(cids_hbm.at[pl.ds(base + j * IDX_CHUNK, IDX_CHUNK)],
                            cid_v.at[j])

        copies = []
        for j in range(n_chunks):
            copies.append(pltpu.async_copy(
                umat_hbm.at[uid_v.at[j]],
                urows_v.at[pl.ds(j * IDX_CHUNK, IDX_CHUNK)], sem))
            copies.append(pltpu.async_copy(
                imat_hbm.at[cid_v.at[j]],
                crows_v.at[pl.ds(j * IDX_CHUNK, IDX_CHUNK)], sem))
        for c in copies:
            c.wait()

        lane_iota = lax.iota(jnp.int32, LANES)
        perms = [lane_iota ^ d for d in (1, 2, 4, 8)]
        lane_eq = [lane_iota == r for r in range(LANES)]
        zeros = jnp.zeros((LANES,), jnp.float32)

        def group_body(g, _):
            row0 = g * LANES
            o = zeros
            for r in range(LANES):
                u0 = urows_v[row0 + r, pl.ds(0, LANES)]
                u1 = urows_v[row0 + r, pl.ds(LANES, LANES)]
                c0 = crows_v[row0 + r, pl.ds(0, LANES)]
                c1 = crows_v[row0 + r, pl.ds(LANES, LANES)]
                s = u0 * c0 + u1 * c1
                for perm in perms:
                    s = s + _take16(s, perm)
                o = jnp.where(lane_eq[r], s, o)
            out_v[pl.ds(row0, LANES)] = 1.0 / (1.0 + jnp.exp(-o))
            return 0

        lax.fori_loop(0, groups, group_body, 0)

        pltpu.sync_copy(out_v, out_hbm.at[pl.ds(base, b_per_w)])

    return sc_kernel


def kernel(user_ids, content_ids, user_matrix, item_matrix):
    batch = user_ids.shape[0]
    dim = user_matrix.shape[1]
    return _build(batch, dim)(user_ids, content_ids, user_matrix, item_matrix)
